# baseline (speedup 1.0000x reference)
import jax
import jax.numpy as jnp
from jax import lax
from jax.experimental import pallas as pl
from jax.experimental.pallas import tpu as pltpu
from jax.experimental.pallas import tpu_sc as plsc

N_CENTERS = 100000
D = 128
B = 16384
MOM = 0.05

NC = 2
NS = 16
NW = NC * NS

H = N_CENTERS // NC
PASSES = 18
C = 2864
SPAN = 192
DUMMY = C
C_PAD = C + 16

RB = B // NW
RCH = 128
SB = B // NS

CPR = 3128
CCH = 136
NCOPY = CPR // CCH
CNT_CH = 3128


def _body(slots, counts, states, labels, counts_rep,
          retr, new_slots, new_counts,
          lab_all, rowids, batchids, ret_idx, rows_v, cbuf, ccbuf,
          st_v, acc_v, slot_v, out_v, cnt_v, cfw_v,
          cblk, cvec, ncout,
          zero_v, ones_v,
          acc_sp, cnt_sp):
  c = lax.axis_index("c")
  s = lax.axis_index("s")
  wid = c * NS + s

  iota = lax.iota(jnp.int32, 16)
  fz = jnp.zeros((16,), jnp.float32)
  fo = jnp.ones((16,), jnp.float32)

  for i in range(16):
    for k in range(D // 16):
      zero_v[i, pl.ds(16 * k, 16)] = fz
      ones_v[i, pl.ds(16 * k, 16)] = fo

  cstart = c * H + jnp.minimum(s * CPR, H - CPR)
  for k in range(NCOPY):
    pltpu.sync_copy(slots.at[pl.ds(cstart + k * CCH, CCH)], cbuf)
    pltpu.sync_copy(cbuf, new_slots.at[pl.ds(cstart + k * CCH, CCH)])
  cnstart = c * H + jnp.minimum(s * CNT_CH, H - CNT_CH)
  pltpu.sync_copy(counts.at[pl.ds(cnstart, CNT_CH)], ccbuf)
  pltpu.sync_copy(ccbuf, new_counts.at[pl.ds(cnstart, CNT_CH)])

  for k in range(RB // RCH):
    off = wid * RB + k * RCH
    pltpu.sync_copy(labels.at[pl.ds(off, RCH)], ret_idx)
    pltpu.sync_copy(slots.at[ret_idx], rows_v)
    pltpu.sync_copy(rows_v, retr.at[pl.ds(off, RCH)])

  pltpu.sync_copy(labels.at[pl.ds(s * SB, SB)], lab_all)

  plsc.subcore_barrier()

  def one_pass(p, carry):
    base = c * H + p * C
    lim = jnp.minimum(C, H - p * C)
    ione = jnp.ones((16,), jnp.int32)
    izero = jnp.zeros((16,), jnp.int32)


    def scan_chunk(j, off):
      lv = lab_all[pl.ds(16 * j, 16)]
      rel = lv - base
      m = (rel >= 0) & (rel < lim)
      mi = jnp.where(m, ione, izero)
      pos = off + plsc.cumsum(mi) - 1
      plsc.store_scatter(rowids, [pos], rel, mask=m)
      plsc.store_scatter(batchids, [pos], iota + (s * SB + 16 * j), mask=m)
      return off + jnp.sum(mi)

    n = lax.fori_loop(0, SB // 16, scan_chunk, jnp.int32(0))
    ngroups = (n + 15) // 16
    glast = jnp.maximum(ngroups - 1, 0)
    lanem = iota < (n - 16 * glast)

    @pl.when(n > 0)
    def _fix_tail():
      lv = rowids[pl.ds(16 * glast, 16)]
      rowids[pl.ds(16 * glast, 16)] = jnp.where(lanem, lv, DUMMY)
      bv = batchids[pl.ds(16 * glast, 16)]
      batchids[pl.ds(16 * glast, 16)] = jnp.where(lanem, bv, 0)

    def zbody(g, cy):
      rv = rowids[pl.ds(16 * g, 16)]
      pltpu.sync_copy(zero_v, acc_sp.at[rv])
      pltpu.sync_copy(zero_v, cnt_sp.at[rv])
      return cy
    lax.fori_loop(0, ngroups, zbody, 0)

    w0 = pl.multiple_of(jnp.minimum(s * SPAN, jnp.maximum(lim - SPAN, 0)), 8)
    for q in range(SPAN // 16):
      pltpu.sync_copy(zero_v,
                      cnt_sp.at[pl.ds(pl.multiple_of(w0 + 16 * q, 8), 16)])

    plsc.subcore_barrier()

    def abody(g, cy):
      rv = rowids[pl.ds(16 * g, 16)]
      bv = batchids[pl.ds(16 * g, 16)]
      pltpu.sync_copy(states.at[bv], st_v)
      pltpu.sync_copy(st_v, acc_sp.at[rv], add=True)
      pltpu.sync_copy(ones_v, cnt_sp.at[rv], add=True)
      return cy
    lax.fori_loop(0, ngroups, abody, 0)

    plsc.subcore_barrier()

    @pl.when(n > 0)
    def _fix_tail_u():
      lv = rowids[pl.ds(16 * glast, 16)]
      mx = plsc.cummax(jnp.where(lanem, lv, jnp.full((16,), -1, jnp.int32)))
      rowids[pl.ds(16 * glast, 16)] = jnp.where(lanem, lv, mx)

    def ubody(g, cy):
      rv = rowids[pl.ds(16 * g, 16)]
      gv = rv + base
      pltpu.sync_copy(acc_sp.at[rv], acc_v)
      pltpu.sync_copy(cnt_sp.at[rv], cnt_v)
      pltpu.sync_copy(slots.at[gv], slot_v)
      pltpu.sync_copy(counts_rep.at[gv], cfw_v)
      fzero = jnp.zeros((16,), jnp.float32)
      fone = jnp.ones((16,), jnp.float32)
      for i in range(16):
        cnt_i = cnt_v[i, pl.ds(0, 16)]
        ci = cfw_v[i, pl.ds(0, 16)]
        fw = ci <= fzero
        inv = fone / cnt_i
        for k in range(D // 16):
          a = acc_v[i, pl.ds(16 * k, 16)]
          sl = slot_v[i, pl.ds(16 * k, 16)]
          mean = a * inv
          out_v[i, pl.ds(16 * k, 16)] = jnp.where(
              fw, mean, sl * (1.0 - MOM) + mean * MOM)
      pltpu.sync_copy(out_v, new_slots.at[gv])
      return cy
    lax.fori_loop(0, ngroups, ubody, 0)

    dn0 = pl.multiple_of(base + w0, 8)
    pltpu.sync_copy(cnt_sp.at[pl.ds(w0, SPAN)], cblk)
    pltpu.sync_copy(counts.at[pl.ds(dn0, SPAN)], cvec)
    for t in range(SPAN // 16):
      dg = plsc.load_gather(cblk, [iota + 16 * t, jnp.zeros((16,), jnp.int32)])
      ncout[pl.ds(16 * t, 16)] = cvec[pl.ds(16 * t, 16)] + dg
    pltpu.sync_copy(ncout, new_counts.at[pl.ds(dn0, SPAN)])

    plsc.subcore_barrier()
    return carry

  lax.fori_loop(0, PASSES, one_pass, 0)


def kernel(slots, counts, states, center_labels, batch_size):
  del batch_size
  out_type = (
      jax.ShapeDtypeStruct((B, D), jnp.float32),
      jax.ShapeDtypeStruct((N_CENTERS, D), jnp.float32),
      jax.ShapeDtypeStruct((N_CENTERS,), jnp.float32),
  )
  scratch = [
      pltpu.VMEM((SB,), jnp.int32),
      pltpu.VMEM((SB + 16,), jnp.int32),
      pltpu.VMEM((SB + 16,), jnp.int32),
      pltpu.VMEM((RCH,), jnp.int32),
      pltpu.VMEM((RCH, D), jnp.float32),
      pltpu.VMEM((CCH, D), jnp.float32),
      pltpu.VMEM((CNT_CH,), jnp.float32),
      pltpu.VMEM((16, D), jnp.float32),
      pltpu.VMEM((16, D), jnp.float32),
      pltpu.VMEM((16, D), jnp.float32),
      pltpu.VMEM((16, D), jnp.float32),
      pltpu.VMEM((16, D), jnp.float32),
      pltpu.VMEM((16, D), jnp.float32),
      pltpu.VMEM((SPAN, D), jnp.float32),
      pltpu.VMEM((SPAN,), jnp.float32),
      pltpu.VMEM((SPAN,), jnp.float32),
      pltpu.VMEM((16, D), jnp.float32),
      pltpu.VMEM((16, D), jnp.float32),
      pltpu.VMEM_SHARED((C_PAD, D), jnp.float32),
      pltpu.VMEM_SHARED((C_PAD, D), jnp.float32),
  ]
  mesh = plsc.VectorSubcoreMesh(
      core_axis_name="c", subcore_axis_name="s", num_cores=NC)
  f = pl.kernel(_body, out_type=out_type, mesh=mesh, scratch_types=scratch,
                compiler_params=pltpu.CompilerParams(needs_layout_passes=False))
  counts_rep = jnp.broadcast_to(counts[:, None], (N_CENTERS, D))
  counts_rep = jnp.asarray(counts_rep)
  return f(slots, counts, states, center_labels, counts_rep)
